# baseline (speedup 1.0000x reference)
import functools

import jax
import jax.numpy as jnp
from jax.experimental import pallas as pl

_TOPK = 64
_A = 128


def _mlp_body(s_ref, g_ref, sj_ref, idx_ref, W1, b1, W2, b2,
              Wf1, bf1, Wf2, bf2, Wf3, bf3, Wf4, bf4, out_ref):
    A = s_ref.shape[0]
    K = sj_ref.shape[1]
    base = pl.program_id(0) * A
    si = s_ref[...]
    sj = sj_ref[...]
    x4 = si[:, None, :] - sj
    row_ids = base + jax.lax.broadcasted_iota(jnp.int32, (A, K), 0)
    eye = (idx_ref[...] == row_ids).astype(jnp.float32)[..., None]
    x5 = jnp.concatenate([x4, eye], axis=-1).reshape(A * K, 5)
    h = jnp.maximum(jnp.dot(x5, W1[...], preferred_element_type=jnp.float32)
                    + b1[...], 0.0)
    h = jnp.maximum(jnp.dot(h, W2[...], preferred_element_type=jnp.float32)
                    + b2[...], 0.0)
    d2 = x4[..., 0] * x4[..., 0] + x4[..., 1] * x4[..., 1]
    mask = (jnp.sqrt(d2) < 1.0).astype(jnp.float32)
    h = h.reshape(A, K, 128) * mask[..., None]
    hm = jnp.max(h, axis=1)
    sg = si[:, :2] - g_ref[...]
    feat = jnp.concatenate([hm, sg, si[:, 2:]], axis=1)
    y = jnp.maximum(jnp.dot(feat, Wf1[...], preferred_element_type=jnp.float32)
                    + bf1[...], 0.0)
    y = jnp.maximum(jnp.dot(y, Wf2[...], preferred_element_type=jnp.float32)
                    + bf2[...], 0.0)
    y = jnp.maximum(jnp.dot(y, Wf3[...], preferred_element_type=jnp.float32)
                    + bf3[...], 0.0)
    y = jnp.dot(y, Wf4[...], preferred_element_type=jnp.float32) + bf4[...]
    y = 2.0 * jax.nn.sigmoid(y) + 0.2
    a_x = -(y[:, 0:1] * sg[:, 0:1] + y[:, 1:2] * si[:, 2:3])
    a_y = -(y[:, 2:3] * sg[:, 1:2] + y[:, 3:4] * si[:, 3:4])
    out_ref[...] = jnp.concatenate([a_x, a_y], axis=1)


def _run_mlp(s, g, sj, idx, W1, b1, W2, b2, Wf1, bf1, Wf2, bf2, Wf3, bf3,
             Wf4, bf4, interpret=False):
    n = s.shape[0]
    K = sj.shape[1]
    grid = (n // _A,)
    blk = lambda *shape: shape
    full = lambda arr: pl.BlockSpec(arr.shape, lambda i: (0,) * arr.ndim)
    in_specs = [
        pl.BlockSpec((_A, 4), lambda i: (i, 0)),
        pl.BlockSpec((_A, 2), lambda i: (i, 0)),
        pl.BlockSpec((_A, K, 4), lambda i: (i, 0, 0)),
        pl.BlockSpec((_A, K), lambda i: (i, 0)),
    ] + [full(w) for w in (W1, b1, W2, b2, Wf1, bf1, Wf2, bf2, Wf3, bf3,
                           Wf4, bf4)]
    return pl.pallas_call(
        _mlp_body,
        grid=grid,
        in_specs=in_specs,
        out_specs=pl.BlockSpec((_A, 2), lambda i: (i, 0)),
        out_shape=jax.ShapeDtypeStruct((n, 2), jnp.float32),
        interpret=interpret,
    )(s, g, sj, idx, W1, b1, W2, b2, Wf1, bf1, Wf2, bf2, Wf3, bf3, Wf4, bf4)


def kernel(s, g, W1, b1, W2, b2, Wf1, bf1, Wf2, bf2, Wf3, bf3, Wf4, bf4,
           interpret=False):
    p = s[:, :2]
    dxy = p[:, None, :] - p[None, :, :]
    key = jnp.sqrt(jnp.sum(dxy * dxy + 1e-6, axis=-1))
    _, idx = jax.lax.top_k(-key, _TOPK)
    sj = jnp.take(s, idx, axis=0)
    b1r = b1.reshape(1, -1)
    b2r = b2.reshape(1, -1)
    bf1r = bf1.reshape(1, -1)
    bf2r = bf2.reshape(1, -1)
    bf3r = bf3.reshape(1, -1)
    bf4r = bf4.reshape(1, -1)
    return _run_mlp(s, g, sj, idx, W1, b1r, W2, b2r, Wf1, bf1r, Wf2, bf2r,
                    Wf3, bf3r, Wf4, bf4r, interpret=interpret)


# speedup vs baseline: 2.5461x; 2.5461x over previous
import functools

import jax
import jax.numpy as jnp
from jax import lax
from jax.experimental import pallas as pl
from jax.experimental.pallas import tpu as pltpu
from jax.experimental.pallas import tpu_sc as plsc

_N = 2048
_C = 128
_CAP = 272
_TRIG = 240
_NW = 32
_AG = _N // _NW
_ATC = 128
_BIGF = 3e38
_PADXY = 1e9


def _sc_select(sT):
    mesh = plsc.VectorSubcoreMesh(core_axis_name="c", subcore_axis_name="s")

    @functools.partial(
        pl.kernel,
        mesh=mesh,
        out_type=jax.ShapeDtypeStruct((5, _N, _C), jnp.float32),
        scratch_types=[
            pltpu.VMEM((_N,), jnp.float32),
            pltpu.VMEM((_N,), jnp.float32),
            pltpu.VMEM((_N,), jnp.float32),
            pltpu.VMEM((_N,), jnp.float32),
            pltpu.VMEM((_CAP,), jnp.float32),
            pltpu.VMEM((_CAP,), jnp.int32),
            pltpu.VMEM((5, _AG, _C), jnp.float32),
            pltpu.SemaphoreType.DMA,
        ],
        compiler_params=pltpu.CompilerParams(needs_layout_passes=False),
    )
    def sel(sT_hbm, out_hbm, sx, sy, svx, svy, ckey, cidx, xout, sem):
        wid = lax.axis_index("s") * 2 + lax.axis_index("c")
        base = wid * _AG
        pltpu.sync_copy(sT_hbm.at[0], sx)
        pltpu.sync_copy(sT_hbm.at[1], sy)
        pltpu.sync_copy(sT_hbm.at[2], svx)
        pltpu.sync_copy(sT_hbm.at[3], svy)
        lanes = lax.broadcasted_iota(jnp.int32, (16,), 0)

        def count_le(thr, cnt):
            def cbody(cc, acc):
                kv = ckey[pl.ds(cc * 16, 16)]
                mm = (kv <= thr) & ((cc * 16 + lanes) < cnt)
                return acc + jnp.sum(mm.astype(jnp.int32))
            return lax.fori_loop(0, _CAP // 16, cbody, jnp.int32(0))

        def shrink(carry):
            cnt, _ = carry

            def srt(cc, u):
                kv = ckey[pl.ds(cc * 16, 16)]
                kv = jnp.where((cc * 16 + lanes) < cnt, kv,
                               jnp.float32(-_BIGF))
                return jnp.maximum(u, lax.sort(kv))

            u = lax.fori_loop(0, _CAP // 16, srt,
                              jnp.full((16,), -_BIGF, jnp.float32))

            def lbs(_, lohi):
                lo, hi = lohi
                k = (lo + hi) // 2
                uk = jnp.sum(jnp.where(lanes == k, u, 0.0))
                ok = count_le(uk, cnt) >= 64
                return (jnp.where(ok, lo, k + 1), jnp.where(ok, k, hi))

            lo, hi = lax.fori_loop(0, 4, lbs, (jnp.int32(0), jnp.int32(15)))
            bound = jnp.sum(jnp.where(lanes == hi, u, 0.0))

            def refil(cc, nc):
                kv = ckey[pl.ds(cc * 16, 16)]
                iv = cidx[pl.ds(cc * 16, 16)]
                mm = (kv <= bound) & ((cc * 16 + lanes) < cnt)
                plsc.store_compressed(ckey.at[pl.ds(nc, 16)], kv, mask=mm)
                plsc.store_compressed(cidx.at[pl.ds(nc, 16)], iv, mask=mm)
                return nc + jnp.sum(mm.astype(jnp.int32))

            nc = lax.fori_loop(0, _CAP // 16, refil, jnp.int32(0))
            return (nc, bound)

        def per_agent(a, _):
            i = base + a
            isplat = i + jnp.zeros((16,), jnp.int32)
            six = plsc.load_gather(sx, [isplat])
            siy = plsc.load_gather(sy, [isplat])
            sivx = plsc.load_gather(svx, [isplat])
            sivy = plsc.load_gather(svy, [isplat])

            def chunk(c, carry):
                cnt, bound = carry
                kx = sx[pl.ds(c * 16, 16)]
                ky = sy[pl.ds(c * 16, 16)]
                dx = six - kx
                dy = siy - ky
                keyv = (dx * dx + 1e-6) + (dy * dy + 1e-6)
                m = keyv <= bound
                plsc.store_compressed(ckey.at[pl.ds(cnt, 16)], keyv, mask=m)
                plsc.store_compressed(cidx.at[pl.ds(cnt, 16)], c * 16 + lanes,
                                      mask=m)
                cnt = cnt + jnp.sum(m.astype(jnp.int32))
                return lax.cond(cnt > _TRIG, shrink, lambda cb: cb,
                                (cnt, bound))

            cnt, bound = lax.fori_loop(
                0, _N // 16, chunk,
                (jnp.int32(0), jnp.float32(1.0 + 3e-6)))

            def w_cond(st):
                cnt, _, tries = st
                return (cnt > _C) & (tries < 8)

            def w_body(st):
                cnt, bound, tries = st
                cnt, bound = shrink((cnt, bound))
                return (cnt, bound, tries + 1)

            cnt, bound, _ = lax.while_loop(w_cond, w_body,
                                           (cnt, bound, jnp.int32(0)))
            cnt = jnp.minimum(cnt, _C)

            for cc in range(_C // 16):
                off = cc * 16
                valid = (off + lanes) < cnt
                iv = cidx[pl.ds(off, 16)]
                iv = jnp.where(valid, iv, i)
                gx = plsc.load_gather(sx, [iv])
                gy = plsc.load_gather(sy, [iv])
                gvx = plsc.load_gather(svx, [iv])
                gvy = plsc.load_gather(svy, [iv])
                xout[0, a, pl.ds(off, 16)] = jnp.where(
                    valid, six - gx, jnp.float32(_PADXY))
                xout[1, a, pl.ds(off, 16)] = jnp.where(
                    valid, siy - gy, jnp.float32(_PADXY))
                xout[2, a, pl.ds(off, 16)] = jnp.where(valid, sivx - gvx, 0.0)
                xout[3, a, pl.ds(off, 16)] = jnp.where(valid, sivy - gvy, 0.0)
                xout[4, a, pl.ds(off, 16)] = jnp.where(
                    valid & (iv == i), 1.0, 0.0)
            return 0

        lax.fori_loop(0, _AG, per_agent, 0)
        for ch in range(5):
            pltpu.sync_copy(xout.at[ch], out_hbm.at[ch, pl.ds(base, _AG)])

    return sel(sT)


def _tc_body(x_ref, sT_ref, gT_ref, W1T, b1, W2T, b2, Wf1T, bf1, Wf2T, bf2,
             Wf3T, bf3, Wf4T, bf4, out_ref):
    A = _ATC
    X = x_ref[...].reshape(5, A * _C)
    dxf = x_ref[0]
    dyf = x_ref[1]
    key = (dxf * dxf + 1e-6) + (dyf * dyf + 1e-6)
    kb = lax.bitcast_convert_type(key, jnp.int32)
    ones = jnp.ones((_C, 1), jnp.float32)

    def bs(_, lohi):
        lo, hi = lohi
        mid = lo + ((hi - lo) >> 1)
        cmp = (kb <= mid).astype(jnp.float32)
        cnt = jnp.dot(cmp, ones, preferred_element_type=jnp.float32)
        ok = cnt >= 64.0
        return (jnp.where(ok, lo, mid + 1), jnp.where(ok, mid, hi))

    lo0 = jnp.zeros((A, 1), jnp.int32)
    hi0 = jnp.full((A, 1), jnp.int32(0x7F7FFFFF))
    _, thr = lax.fori_loop(0, 31, bs, (lo0, hi0))
    mtop = kb <= thr
    mrad = jnp.sqrt(dxf * dxf + dyf * dyf) < 1.0
    msk = (mtop & mrad).astype(jnp.float32)

    h = jnp.maximum(jnp.dot(W1T[...], X,
                            preferred_element_type=jnp.float32) + b1[...], 0.0)
    h = jnp.maximum(jnp.dot(W2T[...], h,
                            preferred_element_type=jnp.float32) + b2[...], 0.0)
    h = h.reshape(128, A, _C) * msk[None, :, :]
    hm = jnp.max(h, axis=2)

    sT = sT_ref[...]
    sg = sT[:2] - gT_ref[...]
    feat = jnp.concatenate([hm, sg, sT[2:]], axis=0)
    y = jnp.maximum(jnp.dot(Wf1T[...], feat,
                            preferred_element_type=jnp.float32) + bf1[...], 0.0)
    y = jnp.maximum(jnp.dot(Wf2T[...], y,
                            preferred_element_type=jnp.float32) + bf2[...], 0.0)
    y = jnp.maximum(jnp.dot(Wf3T[...], y,
                            preferred_element_type=jnp.float32) + bf3[...], 0.0)
    y = jnp.dot(Wf4T[...], y,
                preferred_element_type=jnp.float32) + bf4[...]
    y = 2.0 * jax.nn.sigmoid(y) + 0.2
    a_x = -(y[0:1] * sg[0:1] + y[1:2] * sT[2:3])
    a_y = -(y[2:3] * sg[1:2] + y[3:4] * sT[3:4])
    out_ref[...] = jnp.concatenate([a_x, a_y], axis=0)


def _tc_stage(xcand, sT, gT, W1T, b1, W2T, b2, Wf1T, bf1, Wf2T, bf2, Wf3T,
              bf3, Wf4T, bf4):
    grid = (_N // _ATC,)
    full = lambda arr: pl.BlockSpec(arr.shape, lambda i: (0,) * arr.ndim)
    in_specs = [
        pl.BlockSpec((5, _ATC, _C), lambda i: (0, i, 0)),
        pl.BlockSpec((4, _ATC), lambda i: (0, i)),
        pl.BlockSpec((2, _ATC), lambda i: (0, i)),
    ] + [full(w) for w in (W1T, b1, W2T, b2, Wf1T, bf1, Wf2T, bf2, Wf3T,
                           bf3, Wf4T, bf4)]
    outT = pl.pallas_call(
        _tc_body,
        grid=grid,
        in_specs=in_specs,
        out_specs=pl.BlockSpec((2, _ATC), lambda i: (0, i)),
        out_shape=jax.ShapeDtypeStruct((2, _N), jnp.float32),
    )(xcand, sT, gT, W1T, b1, W2T, b2, Wf1T, bf1, Wf2T, bf2, Wf3T, bf3,
      Wf4T, bf4)
    return outT


def kernel(s, g, W1, b1, W2, b2, Wf1, bf1, Wf2, bf2, Wf3, bf3, Wf4, bf4):
    sT = s.T
    gT = g.T
    xcand = _sc_select(sT)
    outT = _tc_stage(
        xcand, sT, gT,
        W1.T, b1.reshape(-1, 1), W2.T, b2.reshape(-1, 1),
        Wf1.T, bf1.reshape(-1, 1), Wf2.T, bf2.reshape(-1, 1),
        Wf3.T, bf3.reshape(-1, 1), Wf4.T, bf4.reshape(-1, 1))
    return outT.T
